# baseline (speedup 1.0000x reference)
import functools
import jax
import jax.numpy as jnp
from jax import lax
from jax.experimental import pallas as pl
from jax.experimental.pallas import tpu as pltpu
from jax.experimental.pallas import tpu_sc as plsc

NC = 2
NS = 16
NW = NC * NS
L = 16

B = 2048
B2 = 1024
IB = 128

_mesh = functools.partial(
    plsc.VectorSubcoreMesh,
    core_axis_name="c", subcore_axis_name="s", num_cores=NC, num_subcores=NS,
)

_params = pltpu.CompilerParams(needs_layout_passes=False)


def _wid():
  return lax.axis_index("s") * NC + lax.axis_index("c")


def _make_phase1(n_pad, e_pad, p):
  nchunk = p // B2

  def body(field_h, head_h, tail_h, p0_h, p1_h, vel_h, len_h, dtv_h,
           ld_h, a_h, b_h, up_h,
           field_v, hb, tb, p0b, p1b, vb, lb, ldb, ab, bb, ub, dtv_v,
           si0, si1, so0, so1):
    wid = _wid()
    pltpu.sync_copy(dtv_h, dtv_v)
    dt_v = dtv_v[...]
    base0 = wid * p
    sem_in = (si0, si1)
    sem_out = (so0, so1)

    def fire_in(c, q):
      base = base0 + c * B2
      pltpu.async_copy(head_h.at[pl.ds(base, B2)], hb.at[q], sem_in[q])
      pltpu.async_copy(tail_h.at[pl.ds(base, B2)], tb.at[q], sem_in[q])
      pltpu.async_copy(p0_h.at[pl.ds(base, B2)], p0b.at[q], sem_in[q])
      pltpu.async_copy(p1_h.at[pl.ds(base, B2)], p1b.at[q], sem_in[q])
      pltpu.async_copy(vel_h.at[pl.ds(base, B2)], vb.at[q], sem_in[q])
      pltpu.async_copy(len_h.at[pl.ds(base, B2)], lb.at[q], sem_in[q])

    def wait_in(q):
      s0 = pl.ds(base0, B2)
      pltpu.make_async_copy(head_h.at[s0], hb.at[q], sem_in[q]).wait()
      pltpu.make_async_copy(tail_h.at[s0], tb.at[q], sem_in[q]).wait()
      pltpu.make_async_copy(p0_h.at[s0], p0b.at[q], sem_in[q]).wait()
      pltpu.make_async_copy(p1_h.at[s0], p1b.at[q], sem_in[q]).wait()
      pltpu.make_async_copy(vel_h.at[s0], vb.at[q], sem_in[q]).wait()
      pltpu.make_async_copy(len_h.at[s0], lb.at[q], sem_in[q]).wait()

    def fire_out(c, q):
      base = base0 + c * B2
      pltpu.async_copy(ldb.at[q], ld_h.at[pl.ds(base, B2)], sem_out[q])
      pltpu.async_copy(ab.at[q], a_h.at[pl.ds(base, B2)], sem_out[q])
      pltpu.async_copy(bb.at[q], b_h.at[pl.ds(base, B2)], sem_out[q])
      pltpu.async_copy(ub.at[q], up_h.at[pl.ds(base, B2)], sem_out[q])

    def wait_out(q):
      s0 = pl.ds(base0, B2)
      pltpu.make_async_copy(ldb.at[q], ld_h.at[s0], sem_out[q]).wait()
      pltpu.make_async_copy(ab.at[q], a_h.at[s0], sem_out[q]).wait()
      pltpu.make_async_copy(bb.at[q], b_h.at[s0], sem_out[q]).wait()
      pltpu.make_async_copy(ub.at[q], up_h.at[s0], sem_out[q]).wait()

    def compute(q):
      def vec(i, carry2):
        s = pl.ds(i * L, L)
        h = hb[q, s]
        t = tb[q, s]
        v = vb[q, s]
        fh = plsc.load_gather(field_v, [h])
        ft = plsc.load_gather(field_v, [t])
        ld = fh - ft
        up = jnp.where(v <= 0.0, p1b[q, s], p0b[q, s])
        c = v * dt_v / lb[q, s]
        high = 0.5 * ((1.0 + c) * ft + (1.0 - c) * fh)
        low = jnp.where(v > 0.0, ft, fh)
        vh = v * high
        a = v * low
        b = vh - a
        inval = (ld == 0.0) | (up < 0)
        ldb[q, s] = ld
        ab[q, s] = jnp.where(inval, vh, a)
        bb[q, s] = jnp.where(inval, 0.0, b)
        ub[q, s] = jnp.maximum(up, 0)
        return carry2

      lax.fori_loop(0, B2 // L, vec, 0)

    fire_in(0, 0)
    fire_in(1, 1)
    pltpu.sync_copy(field_h, field_v)

    def pair(g, carry):
      c = 2 * g
      wait_in(0)

      @pl.when(c >= 2)
      def _():
        wait_out(0)

      compute(0)
      fire_out(c, 0)

      @pl.when(c + 2 < nchunk)
      def _():
        fire_in(c + 2, 0)

      wait_in(1)

      @pl.when(c >= 2)
      def _():
        wait_out(1)

      compute(1)
      fire_out(c + 1, 1)

      @pl.when(c + 3 < nchunk)
      def _():
        fire_in(c + 3, 1)

      return carry

    lax.fori_loop(0, nchunk // 2, pair, 0)
    wait_out(0)
    wait_out(1)

  f32 = jnp.float32
  i32 = jnp.int32
  out_type = [
      jax.ShapeDtypeStruct((e_pad,), f32),
      jax.ShapeDtypeStruct((e_pad,), f32),
      jax.ShapeDtypeStruct((e_pad,), f32),
      jax.ShapeDtypeStruct((e_pad,), i32),
  ]
  scratch = [
      pltpu.VMEM((n_pad,), f32),
      pltpu.VMEM((2, B2), i32), pltpu.VMEM((2, B2), i32),
      pltpu.VMEM((2, B2), i32), pltpu.VMEM((2, B2), i32),
      pltpu.VMEM((2, B2), f32), pltpu.VMEM((2, B2), f32),
      pltpu.VMEM((2, B2), f32), pltpu.VMEM((2, B2), f32),
      pltpu.VMEM((2, B2), f32), pltpu.VMEM((2, B2), i32),
      pltpu.VMEM((L,), f32),
      pltpu.SemaphoreType.DMA, pltpu.SemaphoreType.DMA,
      pltpu.SemaphoreType.DMA, pltpu.SemaphoreType.DMA,
  ]
  return pl.kernel(body, out_type=out_type, mesh=_mesh(),
                   scratch_types=scratch, compiler_params=_params,
                   name="tvd_phase1")


def _make_phase2(n_pad, e_pad, p, skew):
  p0 = (int(2 * p * skew) // (2 * B2)) * (2 * B2)
  p1 = 2 * p - p0
  nck = (p0 // B2, p1 // B2)

  def body(ld_h, a_h, b_h, up_h, head_h, tail_h,
           part_h,
           acc, ldb, ab, bb, ub, lub, hb, tb,
           sl0, sl1, si0, si1):
    wid = _wid()
    cid = lax.axis_index("c")
    sid = lax.axis_index("s")
    base0 = jnp.where(cid == 0, sid * p0, NS * p0 + sid * p1)
    nchunk = jnp.where(cid == 0, nck[0], nck[1])
    sem_lin = (sl0, sl1)
    sem_idx = (si0, si1)

    def fire_lin(c, q):
      base = pl.multiple_of(base0 + c * B2, 8)
      pltpu.async_copy(ld_h.at[pl.ds(base, B2)], ldb.at[q], sem_lin[q])
      pltpu.async_copy(a_h.at[pl.ds(base, B2)], ab.at[q], sem_lin[q])
      pltpu.async_copy(b_h.at[pl.ds(base, B2)], bb.at[q], sem_lin[q])
      pltpu.async_copy(up_h.at[pl.ds(base, B2)], ub.at[q], sem_lin[q])
      pltpu.async_copy(head_h.at[pl.ds(base, B2)], hb.at[q], sem_lin[q])
      pltpu.async_copy(tail_h.at[pl.ds(base, B2)], tb.at[q], sem_lin[q])

    def wait_lin(q):
      s0 = pl.ds(0, B2)
      pltpu.make_async_copy(ld_h.at[s0], ldb.at[q], sem_lin[q]).wait()
      pltpu.make_async_copy(a_h.at[s0], ab.at[q], sem_lin[q]).wait()
      pltpu.make_async_copy(b_h.at[s0], bb.at[q], sem_lin[q]).wait()
      pltpu.make_async_copy(up_h.at[s0], ub.at[q], sem_lin[q]).wait()
      pltpu.make_async_copy(head_h.at[s0], hb.at[q], sem_lin[q]).wait()
      pltpu.make_async_copy(tail_h.at[s0], tb.at[q], sem_lin[q]).wait()

    def fire_idx(q):
      for j in range(B2 // IB):
        pltpu.async_copy(ld_h.at[ub.at[q].at[pl.ds(j * IB, IB)]],
                         lub.at[q].at[pl.ds(j * IB, IB)], sem_idx[q])

    def wait_idx(q):
      for j in range(B2 // IB):
        pltpu.make_async_copy(ld_h.at[ub.at[q].at[pl.ds(j * IB, IB)]],
                              lub.at[q].at[pl.ds(j * IB, IB)],
                              sem_idx[q]).wait()

    def compute(q):
      def vec(i, carry2):
        s = pl.ds(i * L, L)
        ld = ldb[q, s]
        den = jnp.where(ld == 0.0, 1.0, ld)
        gr = lub[q, s] / den
        ag = jnp.abs(gr)
        fl = (gr + ag) / (1.0 + ag)
        flux = ab[q, s] + fl * bb[q, s]
        plsc.addupdate_scatter(acc, [tb[q, s]], flux)
        plsc.addupdate_scatter(acc, [hb[q, s]], -flux)
        return carry2

      lax.fori_loop(0, B2 // L, vec, 0)

    fire_lin(0, 0)

    def zinit(i, carry):
      acc[pl.ds(i * L, L)] = jnp.zeros((L,), jnp.float32)
      return carry

    lax.fori_loop(0, n_pad // L, zinit, 0)
    wait_lin(0)
    fire_idx(0)
    fire_lin(1, 1)

    def pair(g, carry):
      c = 2 * g
      wait_lin(1)
      fire_idx(1)

      wait_idx(0)
      compute(0)

      @pl.when(c + 2 < nchunk)
      def _():
        fire_lin(c + 2, 0)
        wait_lin(0)
        fire_idx(0)

      wait_idx(1)
      compute(1)

      @pl.when(c + 3 < nchunk)
      def _():
        fire_lin(c + 3, 1)

      return carry

    lax.fori_loop(0, nchunk // 2, pair, 0)
    pltpu.sync_copy(acc, part_h.at[wid])

  f32 = jnp.float32
  i32 = jnp.int32
  out_type = [jax.ShapeDtypeStruct((NW, n_pad), f32)]
  scratch = [
      pltpu.VMEM((n_pad,), f32),
      pltpu.VMEM((2, B2), f32), pltpu.VMEM((2, B2), f32),
      pltpu.VMEM((2, B2), f32),
      pltpu.VMEM((2, B2), i32),
      pltpu.VMEM((2, B2), f32),
      pltpu.VMEM((2, B2), i32), pltpu.VMEM((2, B2), i32),
      pltpu.SemaphoreType.DMA, pltpu.SemaphoreType.DMA,
      pltpu.SemaphoreType.DMA, pltpu.SemaphoreType.DMA,
  ]
  return pl.kernel(body, out_type=out_type, mesh=_mesh(),
                   scratch_types=scratch, compiler_params=_params,
                   name="tvd_phase2")


def _make_phase3(n_pad, span):
  def body(part_h, field_h, area_h, dtv_h, out_h,
           pb, fv, av, ov, dtv_v):
    base = pl.multiple_of(_wid() * span, 128)
    pltpu.sync_copy(field_h.at[pl.ds(base, span)], fv)
    pltpu.sync_copy(area_h.at[pl.ds(base, span)], av)
    pltpu.sync_copy(dtv_h, dtv_v)
    pltpu.sync_copy(part_h.at[:, pl.ds(base, span)], pb)
    dt_v = dtv_v[...]

    def vec(i, carry):
      s = pl.ds(i * L, L)
      acc = pb[0, s]
      for j in range(1, NW):
        acc = acc + pb[j, s]
      ov[s] = fv[s] - dt_v * acc / av[s]
      return carry

    lax.fori_loop(0, span // L, vec, 0)
    pltpu.sync_copy(ov, out_h.at[pl.ds(base, span)])

  f32 = jnp.float32
  out_type = [jax.ShapeDtypeStruct((NW * span,), f32)]
  scratch = [
      pltpu.VMEM((NW, span), f32), pltpu.VMEM((span,), f32),
      pltpu.VMEM((span,), f32), pltpu.VMEM((span,), f32),
      pltpu.VMEM((L,), f32),
  ]
  return pl.kernel(body, out_type=out_type, mesh=_mesh(),
                   scratch_types=scratch, name="tvd_phase3")


def _ceil_to(x, m):
  return ((x + m - 1) // m) * m


def kernel(field, velocity, node_at_link_head, node_at_link_tail,
           parallel_links_at_link, length_of_link, cell_area_at_node, dt):
  n = field.shape[0]
  e = velocity.shape[0]
  i32 = jnp.int32
  f32 = jnp.float32

  p = _ceil_to(-(-e // NW), B)
  e_pad = NW * p
  span = _ceil_to(-(-n // NW), 128)
  n_pad = NW * span

  ep = e_pad - e
  np_ = n_pad - n
  head = jnp.pad(node_at_link_head.astype(i32), (0, ep))
  tail = jnp.pad(node_at_link_tail.astype(i32), (0, ep))
  p0 = jnp.pad(parallel_links_at_link[:, 0].astype(i32), (0, ep),
               constant_values=-1)
  p1 = jnp.pad(parallel_links_at_link[:, 1].astype(i32), (0, ep),
               constant_values=-1)
  vel = jnp.pad(velocity.astype(f32), (0, ep))
  lol = jnp.pad(length_of_link.astype(f32), (0, ep), constant_values=1.0)
  fld = jnp.pad(field.astype(f32), (0, np_))
  area = jnp.pad(cell_area_at_node.astype(f32), (0, np_), constant_values=1.0)
  dtv = jnp.full((L,), dt, dtype=f32)

  ld, a, b, up = _make_phase1(n_pad, e_pad, p)(
      fld, head, tail, p0, p1, vel, lol, dtv)
  (part,) = _make_phase2(n_pad, e_pad, p, 0.42)(ld, a, b, up, head, tail)
  (out,) = _make_phase3(n_pad, span)(part, fld, area, dtv)
  return out[:n]


# speedup vs baseline: 1.0003x; 1.0003x over previous
import functools
import jax
import jax.numpy as jnp
from jax import lax
from jax.experimental import pallas as pl
from jax.experimental.pallas import tpu as pltpu
from jax.experimental.pallas import tpu_sc as plsc

NC = 2
NS = 16
NW = NC * NS
L = 16

B = 2048
B2 = 1024
IB = 128

_mesh = functools.partial(
    plsc.VectorSubcoreMesh,
    core_axis_name="c", subcore_axis_name="s", num_cores=NC, num_subcores=NS,
)

_params = pltpu.CompilerParams(needs_layout_passes=False)


def _wid():
  return lax.axis_index("s") * NC + lax.axis_index("c")


def _make_phase1(n_pad, e_pad, p):
  nchunk = p // B2

  def body(field_h, head_h, tail_h, p0_h, p1_h, vel_h, len_h, dtv_h,
           ld_h, a_h, b_h, up_h,
           field_v, hb, tb, p0b, p1b, vb, lb, ldb, ab, bb, ub, dtv_v,
           si0, si1, so0, so1):
    wid = _wid()
    pltpu.sync_copy(dtv_h, dtv_v)
    dt_v = dtv_v[...]
    base0 = wid * p
    sem_in = (si0, si1)
    sem_out = (so0, so1)

    def fire_in(c, q):
      base = base0 + c * B2
      pltpu.async_copy(head_h.at[pl.ds(base, B2)], hb.at[q], sem_in[q])
      pltpu.async_copy(tail_h.at[pl.ds(base, B2)], tb.at[q], sem_in[q])
      pltpu.async_copy(p0_h.at[pl.ds(base, B2)], p0b.at[q], sem_in[q])
      pltpu.async_copy(p1_h.at[pl.ds(base, B2)], p1b.at[q], sem_in[q])
      pltpu.async_copy(vel_h.at[pl.ds(base, B2)], vb.at[q], sem_in[q])
      pltpu.async_copy(len_h.at[pl.ds(base, B2)], lb.at[q], sem_in[q])

    def wait_in(q):
      s0 = pl.ds(base0, B2)
      pltpu.make_async_copy(head_h.at[s0], hb.at[q], sem_in[q]).wait()
      pltpu.make_async_copy(tail_h.at[s0], tb.at[q], sem_in[q]).wait()
      pltpu.make_async_copy(p0_h.at[s0], p0b.at[q], sem_in[q]).wait()
      pltpu.make_async_copy(p1_h.at[s0], p1b.at[q], sem_in[q]).wait()
      pltpu.make_async_copy(vel_h.at[s0], vb.at[q], sem_in[q]).wait()
      pltpu.make_async_copy(len_h.at[s0], lb.at[q], sem_in[q]).wait()

    def fire_out(c, q):
      base = base0 + c * B2
      pltpu.async_copy(ldb.at[q], ld_h.at[pl.ds(base, B2)], sem_out[q])
      pltpu.async_copy(ab.at[q], a_h.at[pl.ds(base, B2)], sem_out[q])
      pltpu.async_copy(bb.at[q], b_h.at[pl.ds(base, B2)], sem_out[q])
      pltpu.async_copy(ub.at[q], up_h.at[pl.ds(base, B2)], sem_out[q])

    def wait_out(q):
      s0 = pl.ds(base0, B2)
      pltpu.make_async_copy(ldb.at[q], ld_h.at[s0], sem_out[q]).wait()
      pltpu.make_async_copy(ab.at[q], a_h.at[s0], sem_out[q]).wait()
      pltpu.make_async_copy(bb.at[q], b_h.at[s0], sem_out[q]).wait()
      pltpu.make_async_copy(ub.at[q], up_h.at[s0], sem_out[q]).wait()

    def compute(q):
      def vec(i, carry2):
        s = pl.ds(i * L, L)
        h = hb[q, s]
        t = tb[q, s]
        v = vb[q, s]
        fh = plsc.load_gather(field_v, [h])
        ft = plsc.load_gather(field_v, [t])
        ld = fh - ft
        up = jnp.where(v <= 0.0, p1b[q, s], p0b[q, s])
        c = v * dt_v / lb[q, s]
        high = 0.5 * ((1.0 + c) * ft + (1.0 - c) * fh)
        low = jnp.where(v > 0.0, ft, fh)
        vh = v * high
        a = v * low
        b = vh - a
        inval = (ld == 0.0) | (up < 0)
        ldb[q, s] = ld
        ab[q, s] = jnp.where(inval, vh, a)
        bb[q, s] = jnp.where(inval, 0.0, b)
        ub[q, s] = jnp.maximum(up, 0)
        return carry2

      lax.fori_loop(0, B2 // L, vec, 0)

    fire_in(0, 0)
    fire_in(1, 1)
    pltpu.sync_copy(field_h, field_v)

    def pair(g, carry):
      c = 2 * g
      wait_in(0)

      @pl.when(c >= 2)
      def _():
        wait_out(0)

      compute(0)
      fire_out(c, 0)

      @pl.when(c + 2 < nchunk)
      def _():
        fire_in(c + 2, 0)

      wait_in(1)

      @pl.when(c >= 2)
      def _():
        wait_out(1)

      compute(1)
      fire_out(c + 1, 1)

      @pl.when(c + 3 < nchunk)
      def _():
        fire_in(c + 3, 1)

      return carry

    lax.fori_loop(0, nchunk // 2, pair, 0)
    wait_out(0)
    wait_out(1)

  f32 = jnp.float32
  i32 = jnp.int32
  out_type = [
      jax.ShapeDtypeStruct((e_pad,), f32),
      jax.ShapeDtypeStruct((e_pad,), f32),
      jax.ShapeDtypeStruct((e_pad,), f32),
      jax.ShapeDtypeStruct((e_pad,), i32),
  ]
  scratch = [
      pltpu.VMEM((n_pad,), f32),
      pltpu.VMEM((2, B2), i32), pltpu.VMEM((2, B2), i32),
      pltpu.VMEM((2, B2), i32), pltpu.VMEM((2, B2), i32),
      pltpu.VMEM((2, B2), f32), pltpu.VMEM((2, B2), f32),
      pltpu.VMEM((2, B2), f32), pltpu.VMEM((2, B2), f32),
      pltpu.VMEM((2, B2), f32), pltpu.VMEM((2, B2), i32),
      pltpu.VMEM((L,), f32),
      pltpu.SemaphoreType.DMA, pltpu.SemaphoreType.DMA,
      pltpu.SemaphoreType.DMA, pltpu.SemaphoreType.DMA,
  ]
  return pl.kernel(body, out_type=out_type, mesh=_mesh(),
                   scratch_types=scratch, compiler_params=_params,
                   name="tvd_phase1")


def _make_phase2(n_pad, e_pad, p, skew):
  p0 = (int(2 * p * skew) // (2 * B2)) * (2 * B2)
  p1 = 2 * p - p0
  nck = (p0 // B2, p1 // B2)

  def body(ld_h, a_h, b_h, up_h, head_h, tail_h,
           part_h,
           acc, ldb, ab, bb, ub, lub, hb, tb,
           sl0, sl1, si0, si1):
    wid = _wid()
    cid = lax.axis_index("c")
    sid = lax.axis_index("s")
    base0 = jnp.where(cid == 0, sid * p0, NS * p0 + sid * p1)
    nchunk = jnp.where(cid == 0, nck[0], nck[1])
    sem_lin = (sl0, sl1)
    sem_idx = (si0, si1)

    def fire_lin(c, q):
      base = pl.multiple_of(base0 + c * B2, 8)
      pltpu.async_copy(ld_h.at[pl.ds(base, B2)], ldb.at[q], sem_lin[q])
      pltpu.async_copy(a_h.at[pl.ds(base, B2)], ab.at[q], sem_lin[q])
      pltpu.async_copy(b_h.at[pl.ds(base, B2)], bb.at[q], sem_lin[q])
      pltpu.async_copy(up_h.at[pl.ds(base, B2)], ub.at[q], sem_lin[q])
      pltpu.async_copy(head_h.at[pl.ds(base, B2)], hb.at[q], sem_lin[q])
      pltpu.async_copy(tail_h.at[pl.ds(base, B2)], tb.at[q], sem_lin[q])

    def wait_lin(q):
      s0 = pl.ds(0, B2)
      pltpu.make_async_copy(ld_h.at[s0], ldb.at[q], sem_lin[q]).wait()
      pltpu.make_async_copy(a_h.at[s0], ab.at[q], sem_lin[q]).wait()
      pltpu.make_async_copy(b_h.at[s0], bb.at[q], sem_lin[q]).wait()
      pltpu.make_async_copy(up_h.at[s0], ub.at[q], sem_lin[q]).wait()
      pltpu.make_async_copy(head_h.at[s0], hb.at[q], sem_lin[q]).wait()
      pltpu.make_async_copy(tail_h.at[s0], tb.at[q], sem_lin[q]).wait()

    def fire_idx(q):
      for j in range(B2 // IB):
        pltpu.async_copy(ld_h.at[ub.at[q].at[pl.ds(j * IB, IB)]],
                         lub.at[q].at[pl.ds(j * IB, IB)], sem_idx[q])

    def wait_idx(q):
      for j in range(B2 // IB):
        pltpu.make_async_copy(ld_h.at[ub.at[q].at[pl.ds(j * IB, IB)]],
                              lub.at[q].at[pl.ds(j * IB, IB)],
                              sem_idx[q]).wait()

    def compute(q):
      def vec(i, carry2):
        s = pl.ds(i * L, L)
        ld = ldb[q, s]
        den = jnp.where(ld == 0.0, 1.0, ld)
        gr = lub[q, s] / den
        ag = jnp.abs(gr)
        fl = (gr + ag) / (1.0 + ag)
        flux = ab[q, s] + fl * bb[q, s]
        plsc.addupdate_scatter(acc, [tb[q, s]], flux)
        plsc.addupdate_scatter(acc, [hb[q, s]], -flux)
        return carry2

      lax.fori_loop(0, B2 // L, vec, 0)

    fire_lin(0, 0)

    def zinit(i, carry):
      acc[pl.ds(i * L, L)] = jnp.zeros((L,), jnp.float32)
      return carry

    lax.fori_loop(0, n_pad // L, zinit, 0)
    wait_lin(0)
    fire_idx(0)
    fire_lin(1, 1)

    def pair(g, carry):
      c = 2 * g
      wait_lin(1)
      fire_idx(1)

      wait_idx(0)
      compute(0)

      @pl.when(c + 2 < nchunk)
      def _():
        fire_lin(c + 2, 0)
        wait_lin(0)
        fire_idx(0)

      wait_idx(1)
      compute(1)

      @pl.when(c + 3 < nchunk)
      def _():
        fire_lin(c + 3, 1)

      return carry

    lax.fori_loop(0, nchunk // 2, pair, 0)
    pltpu.sync_copy(acc, part_h.at[wid])

  f32 = jnp.float32
  i32 = jnp.int32
  out_type = [jax.ShapeDtypeStruct((NW, n_pad), f32)]
  scratch = [
      pltpu.VMEM((n_pad,), f32),
      pltpu.VMEM((2, B2), f32), pltpu.VMEM((2, B2), f32),
      pltpu.VMEM((2, B2), f32),
      pltpu.VMEM((2, B2), i32),
      pltpu.VMEM((2, B2), f32),
      pltpu.VMEM((2, B2), i32), pltpu.VMEM((2, B2), i32),
      pltpu.SemaphoreType.DMA, pltpu.SemaphoreType.DMA,
      pltpu.SemaphoreType.DMA, pltpu.SemaphoreType.DMA,
  ]
  return pl.kernel(body, out_type=out_type, mesh=_mesh(),
                   scratch_types=scratch, compiler_params=_params,
                   name="tvd_phase2")


def _make_phase3(n_pad, span):
  def body(part_h, field_h, area_h, dtv_h, out_h,
           pb, fv, av, ov, dtv_v):
    base = pl.multiple_of(_wid() * span, 128)
    pltpu.sync_copy(field_h.at[pl.ds(base, span)], fv)
    pltpu.sync_copy(area_h.at[pl.ds(base, span)], av)
    pltpu.sync_copy(dtv_h, dtv_v)
    pltpu.sync_copy(part_h.at[:, pl.ds(base, span)], pb)
    dt_v = dtv_v[...]

    def vec(i, carry):
      s = pl.ds(i * L, L)
      acc = pb[0, s]
      for j in range(1, NW):
        acc = acc + pb[j, s]
      ov[s] = fv[s] - dt_v * acc / av[s]
      return carry

    lax.fori_loop(0, span // L, vec, 0)
    pltpu.sync_copy(ov, out_h.at[pl.ds(base, span)])

  f32 = jnp.float32
  out_type = [jax.ShapeDtypeStruct((NW * span,), f32)]
  scratch = [
      pltpu.VMEM((NW, span), f32), pltpu.VMEM((span,), f32),
      pltpu.VMEM((span,), f32), pltpu.VMEM((span,), f32),
      pltpu.VMEM((L,), f32),
  ]
  return pl.kernel(body, out_type=out_type, mesh=_mesh(),
                   scratch_types=scratch, name="tvd_phase3")


def _ceil_to(x, m):
  return ((x + m - 1) // m) * m


def kernel(field, velocity, node_at_link_head, node_at_link_tail,
           parallel_links_at_link, length_of_link, cell_area_at_node, dt):
  n = field.shape[0]
  e = velocity.shape[0]
  i32 = jnp.int32
  f32 = jnp.float32

  p = _ceil_to(-(-e // NW), B)
  e_pad = NW * p
  span = _ceil_to(-(-n // NW), 128)
  n_pad = NW * span

  ep = e_pad - e
  np_ = n_pad - n
  head = jnp.pad(node_at_link_head.astype(i32), (0, ep))
  tail = jnp.pad(node_at_link_tail.astype(i32), (0, ep))
  p0 = jnp.pad(parallel_links_at_link[:, 0].astype(i32), (0, ep),
               constant_values=-1)
  p1 = jnp.pad(parallel_links_at_link[:, 1].astype(i32), (0, ep),
               constant_values=-1)
  vel = jnp.pad(velocity.astype(f32), (0, ep))
  lol = jnp.pad(length_of_link.astype(f32), (0, ep), constant_values=1.0)
  fld = jnp.pad(field.astype(f32), (0, np_))
  area = jnp.pad(cell_area_at_node.astype(f32), (0, np_), constant_values=1.0)
  dtv = jnp.full((L,), dt, dtype=f32)

  ld, a, b, up = _make_phase1(n_pad, e_pad, p)(
      fld, head, tail, p0, p1, vel, lol, dtv)
  (part,) = _make_phase2(n_pad, e_pad, p, 0.5)(ld, a, b, up, head, tail)
  (out,) = _make_phase3(n_pad, span)(part, fld, area, dtv)
  return out[:n]


# speedup vs baseline: 1.0006x; 1.0003x over previous
import functools
import jax
import jax.numpy as jnp
from jax import lax
from jax.experimental import pallas as pl
from jax.experimental.pallas import tpu as pltpu
from jax.experimental.pallas import tpu_sc as plsc

NC = 2
NS = 16
NW = NC * NS
L = 16

B = 2048
B2 = 1024
IB = 128

_mesh = functools.partial(
    plsc.VectorSubcoreMesh,
    core_axis_name="c", subcore_axis_name="s", num_cores=NC, num_subcores=NS,
)

_params = pltpu.CompilerParams(needs_layout_passes=False)


def _wid():
  return lax.axis_index("s") * NC + lax.axis_index("c")


def _make_phase1(n_pad, e_pad, p):
  nchunk = p // B2

  def body(field_h, head_h, tail_h, p0_h, p1_h, vel_h, len_h, dtv_h,
           ld_h, a_h, b_h, up_h,
           field_v, hb, tb, p0b, p1b, vb, lb, ldb, ab, bb, ub, dtv_v,
           si0, si1, so0, so1):
    wid = _wid()
    pltpu.sync_copy(dtv_h, dtv_v)
    dt_v = dtv_v[...]
    base0 = wid * p
    sem_in = (si0, si1)
    sem_out = (so0, so1)

    def fire_in(c, q):
      base = base0 + c * B2
      pltpu.async_copy(head_h.at[pl.ds(base, B2)], hb.at[q], sem_in[q])
      pltpu.async_copy(tail_h.at[pl.ds(base, B2)], tb.at[q], sem_in[q])
      pltpu.async_copy(p0_h.at[pl.ds(base, B2)], p0b.at[q], sem_in[q])
      pltpu.async_copy(p1_h.at[pl.ds(base, B2)], p1b.at[q], sem_in[q])
      pltpu.async_copy(vel_h.at[pl.ds(base, B2)], vb.at[q], sem_in[q])
      pltpu.async_copy(len_h.at[pl.ds(base, B2)], lb.at[q], sem_in[q])

    def wait_in(q):
      s0 = pl.ds(base0, B2)
      pltpu.make_async_copy(head_h.at[s0], hb.at[q], sem_in[q]).wait()
      pltpu.make_async_copy(tail_h.at[s0], tb.at[q], sem_in[q]).wait()
      pltpu.make_async_copy(p0_h.at[s0], p0b.at[q], sem_in[q]).wait()
      pltpu.make_async_copy(p1_h.at[s0], p1b.at[q], sem_in[q]).wait()
      pltpu.make_async_copy(vel_h.at[s0], vb.at[q], sem_in[q]).wait()
      pltpu.make_async_copy(len_h.at[s0], lb.at[q], sem_in[q]).wait()

    def fire_out(c, q):
      base = base0 + c * B2
      pltpu.async_copy(ldb.at[q], ld_h.at[pl.ds(base, B2)], sem_out[q])
      pltpu.async_copy(ab.at[q], a_h.at[pl.ds(base, B2)], sem_out[q])
      pltpu.async_copy(bb.at[q], b_h.at[pl.ds(base, B2)], sem_out[q])
      pltpu.async_copy(ub.at[q], up_h.at[pl.ds(base, B2)], sem_out[q])

    def wait_out(q):
      s0 = pl.ds(base0, B2)
      pltpu.make_async_copy(ldb.at[q], ld_h.at[s0], sem_out[q]).wait()
      pltpu.make_async_copy(ab.at[q], a_h.at[s0], sem_out[q]).wait()
      pltpu.make_async_copy(bb.at[q], b_h.at[s0], sem_out[q]).wait()
      pltpu.make_async_copy(ub.at[q], up_h.at[s0], sem_out[q]).wait()

    def compute(q):
      def vec(i, carry2):
        s = pl.ds(i * L, L)
        h = hb[q, s]
        t = tb[q, s]
        v = vb[q, s]
        fh = plsc.load_gather(field_v, [h])
        ft = plsc.load_gather(field_v, [t])
        ld = fh - ft
        up = jnp.where(v <= 0.0, p1b[q, s], p0b[q, s])
        c = v * dt_v / lb[q, s]
        high = 0.5 * ((1.0 + c) * ft + (1.0 - c) * fh)
        low = jnp.where(v > 0.0, ft, fh)
        vh = v * high
        a = v * low
        b = vh - a
        inval = (ld == 0.0) | (up < 0)
        ldb[q, s] = ld
        ab[q, s] = jnp.where(inval, vh, a)
        bb[q, s] = jnp.where(inval, 0.0, b)
        ub[q, s] = jnp.maximum(up, 0)
        return carry2

      lax.fori_loop(0, B2 // L, vec, 0)

    fire_in(0, 0)
    fire_in(1, 1)
    pltpu.sync_copy(field_h, field_v)

    def pair(g, carry):
      c = 2 * g
      wait_in(0)

      @pl.when(c >= 2)
      def _():
        wait_out(0)

      compute(0)
      fire_out(c, 0)

      @pl.when(c + 2 < nchunk)
      def _():
        fire_in(c + 2, 0)

      wait_in(1)

      @pl.when(c >= 2)
      def _():
        wait_out(1)

      compute(1)
      fire_out(c + 1, 1)

      @pl.when(c + 3 < nchunk)
      def _():
        fire_in(c + 3, 1)

      return carry

    lax.fori_loop(0, nchunk // 2, pair, 0)
    wait_out(0)
    wait_out(1)

  f32 = jnp.float32
  i32 = jnp.int32
  out_type = [
      jax.ShapeDtypeStruct((e_pad,), f32),
      jax.ShapeDtypeStruct((e_pad,), f32),
      jax.ShapeDtypeStruct((e_pad,), f32),
      jax.ShapeDtypeStruct((e_pad,), i32),
  ]
  scratch = [
      pltpu.VMEM((n_pad,), f32),
      pltpu.VMEM((2, B2), i32), pltpu.VMEM((2, B2), i32),
      pltpu.VMEM((2, B2), i32), pltpu.VMEM((2, B2), i32),
      pltpu.VMEM((2, B2), f32), pltpu.VMEM((2, B2), f32),
      pltpu.VMEM((2, B2), f32), pltpu.VMEM((2, B2), f32),
      pltpu.VMEM((2, B2), f32), pltpu.VMEM((2, B2), i32),
      pltpu.VMEM((L,), f32),
      pltpu.SemaphoreType.DMA, pltpu.SemaphoreType.DMA,
      pltpu.SemaphoreType.DMA, pltpu.SemaphoreType.DMA,
  ]
  return pl.kernel(body, out_type=out_type, mesh=_mesh(),
                   scratch_types=scratch, compiler_params=_params,
                   name="tvd_phase1")


def _make_phase2(n_pad, e_pad, p, skew):
  p0 = (int(2 * p * skew) // (2 * B2)) * (2 * B2)
  p1 = 2 * p - p0
  nck = (p0 // B2, p1 // B2)

  def body(ld_h, a_h, b_h, up_h, head_h, tail_h,
           part_h,
           acc, ldb, ab, bb, ub, lub, hb, tb,
           sl0, sl1, si0, si1):
    wid = _wid()
    cid = lax.axis_index("c")
    sid = lax.axis_index("s")
    base0 = jnp.where(cid == 0, sid * p0, NS * p0 + sid * p1)
    nchunk = jnp.where(cid == 0, nck[0], nck[1])
    sem_lin = (sl0, sl1)
    sem_idx = (si0, si1)

    def fire_lin(c, q):
      base = pl.multiple_of(base0 + c * B2, 8)
      pltpu.async_copy(ld_h.at[pl.ds(base, B2)], ldb.at[q], sem_lin[q])
      pltpu.async_copy(a_h.at[pl.ds(base, B2)], ab.at[q], sem_lin[q])
      pltpu.async_copy(b_h.at[pl.ds(base, B2)], bb.at[q], sem_lin[q])
      pltpu.async_copy(up_h.at[pl.ds(base, B2)], ub.at[q], sem_lin[q])
      pltpu.async_copy(head_h.at[pl.ds(base, B2)], hb.at[q], sem_lin[q])
      pltpu.async_copy(tail_h.at[pl.ds(base, B2)], tb.at[q], sem_lin[q])

    def wait_lin(q):
      s0 = pl.ds(0, B2)
      pltpu.make_async_copy(ld_h.at[s0], ldb.at[q], sem_lin[q]).wait()
      pltpu.make_async_copy(a_h.at[s0], ab.at[q], sem_lin[q]).wait()
      pltpu.make_async_copy(b_h.at[s0], bb.at[q], sem_lin[q]).wait()
      pltpu.make_async_copy(up_h.at[s0], ub.at[q], sem_lin[q]).wait()
      pltpu.make_async_copy(head_h.at[s0], hb.at[q], sem_lin[q]).wait()
      pltpu.make_async_copy(tail_h.at[s0], tb.at[q], sem_lin[q]).wait()

    def fire_idx(q):
      for j in range(B2 // IB):
        pltpu.async_copy(ld_h.at[ub.at[q].at[pl.ds(j * IB, IB)]],
                         lub.at[q].at[pl.ds(j * IB, IB)], sem_idx[q])

    def wait_idx(q):
      for j in range(B2 // IB):
        pltpu.make_async_copy(ld_h.at[ub.at[q].at[pl.ds(j * IB, IB)]],
                              lub.at[q].at[pl.ds(j * IB, IB)],
                              sem_idx[q]).wait()

    def compute(q):
      def vec(i, carry2):
        s = pl.ds(i * L, L)
        ld = ldb[q, s]
        den = jnp.where(ld == 0.0, 1.0, ld)
        gr = lub[q, s] / den
        ag = jnp.abs(gr)
        fl = (gr + ag) / (1.0 + ag)
        flux = ab[q, s] + fl * bb[q, s]
        plsc.addupdate_scatter(acc, [tb[q, s]], flux)
        plsc.addupdate_scatter(acc, [hb[q, s]], -flux)
        return carry2

      lax.fori_loop(0, B2 // L, vec, 0)

    fire_lin(0, 0)

    def zinit(i, carry):
      acc[pl.ds(i * L, L)] = jnp.zeros((L,), jnp.float32)
      return carry

    lax.fori_loop(0, n_pad // L, zinit, 0)
    wait_lin(0)
    fire_idx(0)
    fire_lin(1, 1)

    def pair(g, carry):
      c = 2 * g
      wait_lin(1)
      fire_idx(1)

      wait_idx(0)
      compute(0)

      @pl.when(c + 2 < nchunk)
      def _():
        fire_lin(c + 2, 0)
        wait_lin(0)
        fire_idx(0)

      wait_idx(1)
      compute(1)

      @pl.when(c + 3 < nchunk)
      def _():
        fire_lin(c + 3, 1)

      return carry

    lax.fori_loop(0, nchunk // 2, pair, 0)
    pltpu.sync_copy(acc, part_h.at[wid])

  f32 = jnp.float32
  i32 = jnp.int32
  out_type = [jax.ShapeDtypeStruct((NW, n_pad), f32)]
  scratch = [
      pltpu.VMEM((n_pad,), f32),
      pltpu.VMEM((2, B2), f32), pltpu.VMEM((2, B2), f32),
      pltpu.VMEM((2, B2), f32),
      pltpu.VMEM((2, B2), i32),
      pltpu.VMEM((2, B2), f32),
      pltpu.VMEM((2, B2), i32), pltpu.VMEM((2, B2), i32),
      pltpu.SemaphoreType.DMA, pltpu.SemaphoreType.DMA,
      pltpu.SemaphoreType.DMA, pltpu.SemaphoreType.DMA,
  ]
  return pl.kernel(body, out_type=out_type, mesh=_mesh(),
                   scratch_types=scratch, compiler_params=_params,
                   name="tvd_phase2")


def _make_phase3(n_pad, span):
  def body(part_h, field_h, area_h, dtv_h, out_h,
           pb, fv, av, ov, dtv_v):
    base = pl.multiple_of(_wid() * span, 128)
    pltpu.sync_copy(field_h.at[pl.ds(base, span)], fv)
    pltpu.sync_copy(area_h.at[pl.ds(base, span)], av)
    pltpu.sync_copy(dtv_h, dtv_v)
    pltpu.sync_copy(part_h.at[:, pl.ds(base, span)], pb)
    dt_v = dtv_v[...]

    def vec(i, carry):
      s = pl.ds(i * L, L)
      acc = pb[0, s]
      for j in range(1, NW):
        acc = acc + pb[j, s]
      ov[s] = fv[s] - dt_v * acc / av[s]
      return carry

    lax.fori_loop(0, span // L, vec, 0)
    pltpu.sync_copy(ov, out_h.at[pl.ds(base, span)])

  f32 = jnp.float32
  out_type = [jax.ShapeDtypeStruct((NW * span,), f32)]
  scratch = [
      pltpu.VMEM((NW, span), f32), pltpu.VMEM((span,), f32),
      pltpu.VMEM((span,), f32), pltpu.VMEM((span,), f32),
      pltpu.VMEM((L,), f32),
  ]
  return pl.kernel(body, out_type=out_type, mesh=_mesh(),
                   scratch_types=scratch, name="tvd_phase3")


def _ceil_to(x, m):
  return ((x + m - 1) // m) * m


def kernel(field, velocity, node_at_link_head, node_at_link_tail,
           parallel_links_at_link, length_of_link, cell_area_at_node, dt):
  n = field.shape[0]
  e = velocity.shape[0]
  i32 = jnp.int32
  f32 = jnp.float32

  p = _ceil_to(-(-e // NW), B)
  e_pad = NW * p
  span = _ceil_to(-(-n // NW), 128)
  n_pad = NW * span

  ep = e_pad - e
  np_ = n_pad - n
  head = jnp.pad(node_at_link_head.astype(i32), (0, ep))
  tail = jnp.pad(node_at_link_tail.astype(i32), (0, ep))
  p0 = jnp.pad(parallel_links_at_link[:, 0].astype(i32), (0, ep),
               constant_values=-1)
  p1 = jnp.pad(parallel_links_at_link[:, 1].astype(i32), (0, ep),
               constant_values=-1)
  vel = jnp.pad(velocity.astype(f32), (0, ep))
  lol = jnp.pad(length_of_link.astype(f32), (0, ep), constant_values=1.0)
  fld = jnp.pad(field.astype(f32), (0, np_))
  area = jnp.pad(cell_area_at_node.astype(f32), (0, np_), constant_values=1.0)
  dtv = jnp.full((L,), dt, dtype=f32)

  ld, a, b, up = _make_phase1(n_pad, e_pad, p)(
      fld, head, tail, p0, p1, vel, lol, dtv)
  (part,) = _make_phase2(n_pad, e_pad, p, 0.577)(ld, a, b, up, head, tail)
  (out,) = _make_phase3(n_pad, span)(part, fld, area, dtv)
  return out[:n]
